# baseline (speedup 1.0000x reference)
import functools

import jax
import jax.numpy as jnp
from jax import lax
from jax.experimental import pallas as pl
from jax.experimental.pallas import tpu as pltpu
from jax.experimental.pallas import tpu_sc as plsc

M = 2048
LCACHE = 1024
D = 1024
NH = 16
DH = 64
E = 16
DFF = 1024
BQ = 256
SLAB = BQ + LCACHE
BT = 128
NPAD = M + E * BT
NBLK = NPAD // BT

_SC_NC, _SC_NS = 2, 16
_NW = _SC_NC * _SC_NS
_TOK_W = M // _NW


def _ln(x, g, b):
    mu = jnp.mean(x, axis=-1, keepdims=True)
    var = jnp.mean((x - mu) ** 2, axis=-1, keepdims=True)
    return (x - mu) * lax.rsqrt(var + 1e-5) * g + b


def _qkv_body(x_ref, wq_ref, wk_ref, wv_ref, q_ref, k_ref, v_ref):
    x = x_ref[...]
    for w_ref, o_ref in ((wq_ref, q_ref), (wk_ref, k_ref), (wv_ref, v_ref)):
        y = jnp.dot(x, w_ref[...], preferred_element_type=jnp.float32)
        for hh in range(NH):
            o_ref[hh] = y[:, hh * DH:(hh + 1) * DH]


def _qkv(h_all, Wq, Wk, Wv):
    n = h_all.shape[0]
    bs_x = pl.BlockSpec((BQ, D), lambda i: (i, 0))
    bs_w = pl.BlockSpec((D, D), lambda i: (0, 0))
    bs_o = pl.BlockSpec((NH, BQ, DH), lambda i: (0, i, 0))
    out = jax.ShapeDtypeStruct((NH, n, DH), jnp.float32)
    return pl.pallas_call(
        _qkv_body,
        grid=(n // BQ,),
        in_specs=[bs_x, bs_w, bs_w, bs_w],
        out_specs=[bs_o, bs_o, bs_o],
        out_shape=[out, out, out],
    )(h_all, Wq, Wk, Wv)


HB = 2


def _attn_body(q_ref, k_ref, v_ref, pe_ref, mb_ref, o_ref):
    i = pl.program_id(1)
    mb = mb_ref[...]
    for hh in range(HB):
        q = q_ref[hh] * 0.125
        pos = jnp.concatenate(
            [jnp.dot(q[g * 16:(g + 1) * 16, :], pe_ref[g],
                     preferred_element_type=jnp.float32)
             for g in range(BQ // 16)], axis=0)
        row = lax.broadcasted_iota(jnp.int32, (BQ, SLAB), 0)
        for b in range(4):
            s = 1 << b
            rolled = jnp.roll(pos, s, axis=1)
            pos = jnp.where((row & s) != 0, rolled, pos)
        ks = k_ref[hh, pl.ds(i * BQ, SLAB), :]
        vs = v_ref[hh, pl.ds(i * BQ, SLAB), :]
        cont = lax.dot_general(q, ks, (((1,), (1,)), ((), ())),
                               preferred_element_type=jnp.float32)
        scores = cont + pos + mb
        m = jnp.max(scores, axis=1, keepdims=True)
        ex = jnp.exp(scores - m)
        probs = ex / jnp.sum(ex, axis=1, keepdims=True)
        o_ref[hh] = jnp.dot(probs, vs, preferred_element_type=jnp.float32)


def _attn(q_all, k_all, v_all, key_pe2d, mask_bias):
    n_all = M + LCACHE
    return pl.pallas_call(
        _attn_body,
        grid=(NH // HB, M // BQ),
        in_specs=[
            pl.BlockSpec((HB, BQ, DH), lambda h, i: (h, i + LCACHE // BQ, 0)),
            pl.BlockSpec((HB, n_all, DH), lambda h, i: (h, 0, 0)),
            pl.BlockSpec((HB, n_all, DH), lambda h, i: (h, 0, 0)),
            pl.BlockSpec((BQ // 16, DH, SLAB), lambda h, i: (0, 0, 0)),
            pl.BlockSpec((BQ, SLAB), lambda h, i: (0, 0)),
        ],
        out_specs=pl.BlockSpec((HB, BQ, DH), lambda h, i: (h, i, 0)),
        out_shape=jax.ShapeDtypeStruct((NH, M, DH), jnp.float32),
    )(q_all, k_all, v_all, key_pe2d, mask_bias)


def _oproj_body(o_ref, wo_ref, h_ref, g_ref, b_ref, out_ref):
    o = jnp.concatenate([o_ref[hh] for hh in range(NH)], axis=1)
    y = jnp.dot(o, wo_ref[...], preferred_element_type=jnp.float32) + h_ref[...]
    out_ref[...] = _ln(y, g_ref[...], b_ref[...])


def _oproj(o, Wo, h2, g, b):
    bs_x = pl.BlockSpec((BQ, D), lambda i: (i, 0))
    bs_w = pl.BlockSpec((D, D), lambda i: (0, 0))
    bs_v = pl.BlockSpec((1, D), lambda i: (0, 0))
    return pl.pallas_call(
        _oproj_body,
        grid=(M // BQ,),
        in_specs=[pl.BlockSpec((NH, BQ, DH), lambda i: (0, i, 0)), bs_w, bs_x, bs_v, bs_v],
        out_specs=bs_x,
        out_shape=jax.ShapeDtypeStruct((M, D), jnp.float32),
    )(o, Wo, h2, g, b)


def _gate_body(x_ref, wg_ref, bg_ref, dest_ref, be_ref, tot_ref):
    logits = jnp.dot(x_ref[...], wg_ref[...],
                     preferred_element_type=jnp.float32) + bg_ref[...]
    mx = jnp.max(logits, axis=1, keepdims=True)
    col = lax.broadcasted_iota(jnp.int32, (M, E), 1)
    eid = jnp.min(jnp.where(logits == mx, col, E), axis=1, keepdims=True)
    H = (col == eid).astype(jnp.float32)
    counts = jnp.sum(H, axis=0, keepdims=True)
    padded = jnp.floor((counts + (BT - 1)) * (1.0 / BT)) * BT
    er = lax.broadcasted_iota(jnp.int32, (E, E), 0)
    ec = lax.broadcasted_iota(jnp.int32, (E, E), 1)
    tri = (er < ec).astype(jnp.float32)
    off = jnp.dot(padded, tri, preferred_element_type=jnp.float32)
    ch = 2 * BT
    for c in range(M // ch):
        Hc = H[c * ch:(c + 1) * ch, :]
        rr = lax.broadcasted_iota(jnp.int32, (ch, M), 0) + c * ch
        cc = lax.broadcasted_iota(jnp.int32, (ch, M), 1)
        tric = (cc < rr).astype(jnp.float32)
        Cc = jnp.dot(tric, H, preferred_element_type=jnp.float32)
        dvals = jnp.sum(Hc * (Cc + off), axis=1, keepdims=True)
        dest_ref[pl.ds(c * ch, ch), :] = dvals.astype(jnp.int32)
    total = jnp.sum(padded)
    bb = lax.broadcasted_iota(jnp.int32, (NBLK, 1), 0).astype(jnp.float32)
    bclamp = jnp.minimum(bb, total * (1.0 / BT) - 1.0)
    cmp = (jnp.broadcast_to(off, (NBLK, E)) <= BT * bclamp).astype(jnp.float32)
    be_ref[...] = (jnp.sum(cmp, axis=1, keepdims=True) - 1.0).astype(jnp.int32)
    tot_ref[...] = jnp.full((1, 1), total * (1.0 / BT), jnp.float32).astype(jnp.int32)


def _gate(x, Wg, bg):
    return pl.pallas_call(
        _gate_body,
        grid=(1,),
        in_specs=[
            pl.BlockSpec((M, D), lambda i: (0, 0)),
            pl.BlockSpec((D, E), lambda i: (0, 0)),
            pl.BlockSpec((1, E), lambda i: (0, 0)),
        ],
        out_specs=[
            pl.BlockSpec((M, 1), lambda i: (0, 0)),
            pl.BlockSpec((NBLK, 1), lambda i: (0, 0)),
            pl.BlockSpec((1, 1), lambda i: (0, 0)),
        ],
        out_shape=[
            jax.ShapeDtypeStruct((M, 1), jnp.int32),
            jax.ShapeDtypeStruct((NBLK, 1), jnp.int32),
            jax.ShapeDtypeStruct((1, 1), jnp.int32),
        ],
    )(x, Wg, bg)


def _dispatch(x, dest):
    mesh = plsc.VectorSubcoreMesh(core_axis_name="c", subcore_axis_name="s")

    @functools.partial(
        pl.kernel,
        out_type=jax.ShapeDtypeStruct((NPAD, D), jnp.float32),
        mesh=mesh,
        scratch_types=[
            pltpu.VMEM((_TOK_W,), jnp.int32),
            pltpu.VMEM((_TOK_W, D), jnp.float32),
            pltpu.SemaphoreType.DMA,
        ],
    )
    def run(x_hbm, dest_hbm, out_hbm, idx_v, rows_v, sem):
        wid = lax.axis_index("s") * _SC_NC + lax.axis_index("c")
        base = wid * _TOK_W
        pltpu.sync_copy(dest_hbm.at[pl.ds(base, _TOK_W)], idx_v)
        pltpu.sync_copy(x_hbm.at[pl.ds(base, _TOK_W)], rows_v)
        pltpu.async_copy(rows_v, out_hbm.at[idx_v], sem).wait()

    return run(x, dest)


def _combine(y_sorted, dest):
    mesh = plsc.VectorSubcoreMesh(core_axis_name="c", subcore_axis_name="s")

    @functools.partial(
        pl.kernel,
        out_type=jax.ShapeDtypeStruct((M, D), jnp.float32),
        mesh=mesh,
        scratch_types=[
            pltpu.VMEM((_TOK_W,), jnp.int32),
            pltpu.VMEM((_TOK_W, D), jnp.float32),
            pltpu.SemaphoreType.DMA,
        ],
    )
    def run(y_hbm, dest_hbm, out_hbm, idx_v, rows_v, sem):
        wid = lax.axis_index("s") * _SC_NC + lax.axis_index("c")
        base = wid * _TOK_W
        pltpu.sync_copy(dest_hbm.at[pl.ds(base, _TOK_W)], idx_v)
        pltpu.async_copy(y_hbm.at[idx_v], rows_v, sem).wait()
        pltpu.sync_copy(rows_v, out_hbm.at[pl.ds(base, _TOK_W)])

    return run(y_sorted, dest)


def _ffn_body(be_ref, tot_ref, x_ref, w1_ref, b1_ref, w2_ref, b2_ref, y_ref):
    bk = pl.program_id(0)

    @pl.when(bk < tot_ref[0])
    def _():
        x = x_ref[...]
        h1 = jnp.maximum(
            jnp.dot(x, w1_ref[0], preferred_element_type=jnp.float32) + b1_ref[0], 0.0)
        y_ref[...] = jnp.dot(h1, w2_ref[0],
                             preferred_element_type=jnp.float32) + b2_ref[0]


def _ffn(be, tot, x_sorted, W1, b1, W2, b2):
    grid_spec = pltpu.PrefetchScalarGridSpec(
        num_scalar_prefetch=2,
        grid=(NBLK,),
        in_specs=[
            pl.BlockSpec((BT, D), lambda bk, be_r, tot_r: (bk, 0)),
            pl.BlockSpec((1, D, DFF), lambda bk, be_r, tot_r: (be_r[bk], 0, 0)),
            pl.BlockSpec((1, 1, DFF), lambda bk, be_r, tot_r: (be_r[bk], 0, 0)),
            pl.BlockSpec((1, DFF, D), lambda bk, be_r, tot_r: (be_r[bk], 0, 0)),
            pl.BlockSpec((1, 1, D), lambda bk, be_r, tot_r: (be_r[bk], 0, 0)),
        ],
        out_specs=pl.BlockSpec((BT, D), lambda bk, be_r, tot_r: (bk, 0)),
    )
    return pl.pallas_call(
        _ffn_body,
        grid_spec=grid_spec,
        out_shape=jax.ShapeDtypeStruct((NPAD, D), jnp.float32),
    )(be, tot, x_sorted, W1, b1.reshape(E, 1, DFF),
      W2, b2.reshape(E, 1, D))


def _final_body(hatt_ref, y_ref, gm_ref, bm_ref, g2_ref, b2_ref, out_ref):
    hatt = hatt_ref[...]
    smoe = _ln(hatt + y_ref[...], gm_ref[...], bm_ref[...])
    out_ref[...] = _ln(hatt + smoe, g2_ref[...], b2_ref[...])


def _final(h_att, y, gm, bm, g2, b2):
    bs_x = pl.BlockSpec((BQ, D), lambda i: (i, 0))
    bs_v = pl.BlockSpec((1, D), lambda i: (0, 0))
    return pl.pallas_call(
        _final_body,
        grid=(M // BQ,),
        in_specs=[bs_x, bs_x, bs_v, bs_v, bs_v, bs_v],
        out_specs=bs_x,
        out_shape=jax.ShapeDtypeStruct((M, D), jnp.float32),
    )(h_att, y, gm, bm, g2, b2)


def kernel(h, h_cache, key_pe, Wq, Wk, Wv, Wo, Wg, bg, W1, b1, W2, b2,
           ln1_g, ln1_b, lnm_g, lnm_b, ln2_g, ln2_b):
    h2 = h[0]
    h_all = jnp.concatenate([h_cache[0], h2], axis=0)
    q_all, k_all, v_all = _qkv(h_all, Wq, Wk, Wv)
    aa = jnp.arange(SLAB)[None, :]
    rr = jnp.arange(BQ)[:, None]
    mask_bias = jnp.where((aa >= rr) & (aa < rr + LCACHE), 0.0, -1e30).astype(jnp.float32)
    gg = jnp.arange(BQ // 16)[:, None]
    cols = (jnp.arange(SLAB)[None, :] - 16 * gg) % LCACHE
    pe_ext = key_pe[0][None, :, :][:, :, cols.reshape(-1)].reshape(
        DH, BQ // 16, SLAB).transpose(1, 0, 2)
    o = _attn(q_all, k_all, v_all, pe_ext, mask_bias)
    h_att = _oproj(o, Wo, h2, ln1_g.reshape(1, D), ln1_b.reshape(1, D))
    dest2d, be2d, tot2d = _gate(h_att, Wg, bg.reshape(1, E))
    dest = dest2d.reshape(M)
    be = be2d.reshape(NBLK)
    tot = tot2d.reshape(1)
    x_sorted = _dispatch(h_att, dest)
    y_sorted = _ffn(be, tot, x_sorted, W1, b1, W2, b2)
    y = _combine(y_sorted, dest)
    out = _final(h_att, y, lnm_g.reshape(1, D), lnm_b.reshape(1, D),
                 ln2_g.reshape(1, D), ln2_b.reshape(1, D))
    return out.reshape(1, M, D)


# speedup vs baseline: 1.2338x; 1.2338x over previous
import functools

import jax
import jax.numpy as jnp
from jax import lax
from jax.experimental import pallas as pl
from jax.experimental.pallas import tpu as pltpu
from jax.experimental.pallas import tpu_sc as plsc

M = 2048
LCACHE = 1024
D = 1024
NH = 16
DH = 64
E = 16
DFF = 1024
BQ = 256
SLAB = BQ + LCACHE
BT = 128
NPAD = M + E * BT
NBLK = NPAD // BT

_SC_NC, _SC_NS = 2, 16
_NW = _SC_NC * _SC_NS
_TOK_W = M // _NW


def _ln(x, g, b):
    mu = jnp.mean(x, axis=-1, keepdims=True)
    var = jnp.mean((x - mu) ** 2, axis=-1, keepdims=True)
    return (x - mu) * lax.rsqrt(var + 1e-5) * g + b


def _qkv_body(x_ref, wq_ref, wk_ref, wv_ref, q_ref, k_ref, v_ref):
    x = x_ref[...]
    for w_ref, o_ref in ((wq_ref, q_ref), (wk_ref, k_ref), (wv_ref, v_ref)):
        y = jnp.dot(x, w_ref[...], preferred_element_type=jnp.float32)
        for hh in range(NH):
            o_ref[hh] = y[:, hh * DH:(hh + 1) * DH]


def _qkv(h_all, Wq, Wk, Wv):
    n = h_all.shape[0]
    bs_x = pl.BlockSpec((BQ, D), lambda i: (i, 0))
    bs_w = pl.BlockSpec((D, D), lambda i: (0, 0))
    bs_o = pl.BlockSpec((NH, BQ, DH), lambda i: (0, i, 0))
    out = jax.ShapeDtypeStruct((NH, n, DH), jnp.float32)
    return pl.pallas_call(
        _qkv_body,
        grid=(n // BQ,),
        in_specs=[bs_x, bs_w, bs_w, bs_w],
        out_specs=[bs_o, bs_o, bs_o],
        out_shape=[out, out, out],
    )(h_all, Wq, Wk, Wv)


def _attn_body(q_ref, k_ref, v_ref, pe_ref, mb_ref, o_ref):
    i = pl.program_id(1)
    q = q_ref[0] * 0.125
    pos = jnp.dot(q, pe_ref[0], preferred_element_type=jnp.float32)
    ks = k_ref[0, pl.ds(i * BQ, SLAB), :]
    vs = v_ref[0, pl.ds(i * BQ, SLAB), :]
    cont = lax.dot_general(q, ks, (((1,), (1,)), ((), ())),
                           preferred_element_type=jnp.float32)
    scores = cont + pos + mb_ref[...]
    m = jnp.max(scores, axis=1, keepdims=True)
    ex = jnp.exp(scores - m)
    probs = ex / jnp.sum(ex, axis=1, keepdims=True)
    o_ref[0] = jnp.dot(probs, vs, preferred_element_type=jnp.float32)


def _attn(q_all, k_all, v_all, key_pe2d, mask_bias):
    n_all = M + LCACHE
    return pl.pallas_call(
        _attn_body,
        grid=(NH, M // BQ),
        in_specs=[
            pl.BlockSpec((1, BQ, DH), lambda h, i: (h, i + LCACHE // BQ, 0)),
            pl.BlockSpec((1, n_all, DH), lambda h, i: (h, 0, 0)),
            pl.BlockSpec((1, n_all, DH), lambda h, i: (h, 0, 0)),
            pl.BlockSpec((BQ // 16, DH, SLAB), lambda h, i: (0, 0, 0)),
            pl.BlockSpec((BQ, SLAB), lambda h, i: (0, 0)),
        ],
        out_specs=pl.BlockSpec((1, BQ, DH), lambda h, i: (h, i, 0)),
        out_shape=jax.ShapeDtypeStruct((NH, M, DH), jnp.float32),
    )(q_all, k_all, v_all, key_pe2d, mask_bias)


def _oproj_body(o_ref, wo_ref, h_ref, g_ref, b_ref, out_ref):
    o = jnp.concatenate([o_ref[hh] for hh in range(NH)], axis=1)
    y = jnp.dot(o, wo_ref[...], preferred_element_type=jnp.float32) + h_ref[...]
    out_ref[...] = _ln(y, g_ref[...], b_ref[...])


def _oproj(o, Wo, h2, g, b):
    bs_x = pl.BlockSpec((BQ, D), lambda i: (i, 0))
    bs_w = pl.BlockSpec((D, D), lambda i: (0, 0))
    bs_v = pl.BlockSpec((1, D), lambda i: (0, 0))
    return pl.pallas_call(
        _oproj_body,
        grid=(M // BQ,),
        in_specs=[pl.BlockSpec((NH, BQ, DH), lambda i: (0, i, 0)), bs_w, bs_x, bs_v, bs_v],
        out_specs=bs_x,
        out_shape=jax.ShapeDtypeStruct((M, D), jnp.float32),
    )(o, Wo, h2, g, b)


def _gate_body(x_ref, wg_ref, bg_ref, dest_ref, be_ref, tot_ref):
    logits = jnp.dot(x_ref[...], wg_ref[...],
                     preferred_element_type=jnp.float32) + bg_ref[...]
    mx = jnp.max(logits, axis=1, keepdims=True)
    col = lax.broadcasted_iota(jnp.int32, (M, E), 1)
    eid = jnp.min(jnp.where(logits == mx, col, E), axis=1, keepdims=True)
    H = (col == eid).astype(jnp.float32)
    counts = jnp.sum(H, axis=0, keepdims=True)
    padded = jnp.floor((counts + (BT - 1)) * (1.0 / BT)) * BT
    er = lax.broadcasted_iota(jnp.int32, (E, E), 0)
    ec = lax.broadcasted_iota(jnp.int32, (E, E), 1)
    tri = (er < ec).astype(jnp.float32)
    off = jnp.dot(padded, tri, preferred_element_type=jnp.float32)
    ch = 2 * BT
    for c in range(M // ch):
        Hc = H[c * ch:(c + 1) * ch, :]
        rr = lax.broadcasted_iota(jnp.int32, (ch, M), 0) + c * ch
        cc = lax.broadcasted_iota(jnp.int32, (ch, M), 1)
        tric = (cc < rr).astype(jnp.float32)
        Cc = jnp.dot(tric, H, preferred_element_type=jnp.float32)
        dvals = jnp.sum(Hc * (Cc + off), axis=1, keepdims=True)
        dest_ref[pl.ds(c * ch, ch), :] = dvals.astype(jnp.int32)
    total = jnp.sum(padded)
    bb = lax.broadcasted_iota(jnp.int32, (NBLK, 1), 0).astype(jnp.float32)
    bclamp = jnp.minimum(bb, total * (1.0 / BT) - 1.0)
    cmp = (jnp.broadcast_to(off, (NBLK, E)) <= BT * bclamp).astype(jnp.float32)
    be_ref[...] = (jnp.sum(cmp, axis=1, keepdims=True) - 1.0).astype(jnp.int32)
    tot_ref[...] = jnp.full((1, 1), total * (1.0 / BT), jnp.float32).astype(jnp.int32)


def _gate(x, Wg, bg):
    return pl.pallas_call(
        _gate_body,
        grid=(1,),
        in_specs=[
            pl.BlockSpec((M, D), lambda i: (0, 0)),
            pl.BlockSpec((D, E), lambda i: (0, 0)),
            pl.BlockSpec((1, E), lambda i: (0, 0)),
        ],
        out_specs=[
            pl.BlockSpec((M, 1), lambda i: (0, 0)),
            pl.BlockSpec((NBLK, 1), lambda i: (0, 0)),
            pl.BlockSpec((1, 1), lambda i: (0, 0)),
        ],
        out_shape=[
            jax.ShapeDtypeStruct((M, 1), jnp.int32),
            jax.ShapeDtypeStruct((NBLK, 1), jnp.int32),
            jax.ShapeDtypeStruct((1, 1), jnp.int32),
        ],
    )(x, Wg, bg)


def _dispatch(x, dest):
    mesh = plsc.VectorSubcoreMesh(core_axis_name="c", subcore_axis_name="s")

    @functools.partial(
        pl.kernel,
        out_type=jax.ShapeDtypeStruct((NPAD, D), jnp.float32),
        mesh=mesh,
        scratch_types=[
            pltpu.VMEM((_TOK_W,), jnp.int32),
            pltpu.VMEM((_TOK_W, D), jnp.float32),
            pltpu.SemaphoreType.DMA,
        ],
    )
    def run(x_hbm, dest_hbm, out_hbm, idx_v, rows_v, sem):
        wid = lax.axis_index("s") * _SC_NC + lax.axis_index("c")
        base = wid * _TOK_W
        pltpu.sync_copy(dest_hbm.at[pl.ds(base, _TOK_W)], idx_v)
        pltpu.sync_copy(x_hbm.at[pl.ds(base, _TOK_W)], rows_v)
        pltpu.async_copy(rows_v, out_hbm.at[idx_v], sem).wait()

    return run(x, dest)


def _combine(y_sorted, dest):
    mesh = plsc.VectorSubcoreMesh(core_axis_name="c", subcore_axis_name="s")

    @functools.partial(
        pl.kernel,
        out_type=jax.ShapeDtypeStruct((M, D), jnp.float32),
        mesh=mesh,
        scratch_types=[
            pltpu.VMEM((_TOK_W,), jnp.int32),
            pltpu.VMEM((_TOK_W, D), jnp.float32),
            pltpu.SemaphoreType.DMA,
        ],
    )
    def run(y_hbm, dest_hbm, out_hbm, idx_v, rows_v, sem):
        wid = lax.axis_index("s") * _SC_NC + lax.axis_index("c")
        base = wid * _TOK_W
        pltpu.sync_copy(dest_hbm.at[pl.ds(base, _TOK_W)], idx_v)
        pltpu.async_copy(y_hbm.at[idx_v], rows_v, sem).wait()
        pltpu.sync_copy(rows_v, out_hbm.at[pl.ds(base, _TOK_W)])

    return run(y_sorted, dest)


def _ffn_body(be_ref, tot_ref, x_ref, w1_ref, b1_ref, w2_ref, b2_ref, y_ref):
    bk = pl.program_id(0)

    @pl.when(bk < tot_ref[0])
    def _():
        x = x_ref[...]
        h1 = jnp.maximum(
            jnp.dot(x, w1_ref[0], preferred_element_type=jnp.float32) + b1_ref[0], 0.0)
        y_ref[...] = jnp.dot(h1, w2_ref[0],
                             preferred_element_type=jnp.float32) + b2_ref[0]


def _ffn(be, tot, x_sorted, W1, b1, W2, b2):
    grid_spec = pltpu.PrefetchScalarGridSpec(
        num_scalar_prefetch=2,
        grid=(NBLK,),
        in_specs=[
            pl.BlockSpec((BT, D), lambda bk, be_r, tot_r: (bk, 0)),
            pl.BlockSpec((1, D, DFF), lambda bk, be_r, tot_r: (be_r[bk], 0, 0)),
            pl.BlockSpec((1, 1, DFF), lambda bk, be_r, tot_r: (be_r[bk], 0, 0)),
            pl.BlockSpec((1, DFF, D), lambda bk, be_r, tot_r: (be_r[bk], 0, 0)),
            pl.BlockSpec((1, 1, D), lambda bk, be_r, tot_r: (be_r[bk], 0, 0)),
        ],
        out_specs=pl.BlockSpec((BT, D), lambda bk, be_r, tot_r: (bk, 0)),
    )
    return pl.pallas_call(
        _ffn_body,
        grid_spec=grid_spec,
        out_shape=jax.ShapeDtypeStruct((NPAD, D), jnp.float32),
    )(be, tot, x_sorted, W1, b1.reshape(E, 1, DFF),
      W2, b2.reshape(E, 1, D))


def _final_body(hatt_ref, y_ref, gm_ref, bm_ref, g2_ref, b2_ref, out_ref):
    hatt = hatt_ref[...]
    smoe = _ln(hatt + y_ref[...], gm_ref[...], bm_ref[...])
    out_ref[...] = _ln(hatt + smoe, g2_ref[...], b2_ref[...])


def _final(h_att, y, gm, bm, g2, b2):
    bs_x = pl.BlockSpec((BQ, D), lambda i: (i, 0))
    bs_v = pl.BlockSpec((1, D), lambda i: (0, 0))
    return pl.pallas_call(
        _final_body,
        grid=(M // BQ,),
        in_specs=[bs_x, bs_x, bs_v, bs_v, bs_v, bs_v],
        out_specs=bs_x,
        out_shape=jax.ShapeDtypeStruct((M, D), jnp.float32),
    )(h_att, y, gm, bm, g2, b2)


def kernel(h, h_cache, key_pe, Wq, Wk, Wv, Wo, Wg, bg, W1, b1, W2, b2,
           ln1_g, ln1_b, lnm_g, lnm_b, ln2_g, ln2_b):
    h2 = h[0]
    h_all = jnp.concatenate([h_cache[0], h2], axis=0)
    q_all, k_all, v_all = _qkv(h_all, Wq, Wk, Wv)
    aa = jnp.arange(SLAB)[None, :]
    rr = jnp.arange(BQ)[:, None]
    mask_bias = jnp.where((aa >= rr) & (aa < rr + LCACHE), 0.0, -1e30).astype(jnp.float32)
    gg = jnp.arange(BQ // 16)[:, None]
    cols = (jnp.arange(SLAB)[None, :] - 16 * gg) % LCACHE
    pe_ext = key_pe[0][None, :, :][:, :, cols.reshape(-1)].reshape(
        DH, BQ // 16, SLAB).transpose(1, 0, 2)
    o = _attn(q_all, k_all, v_all, pe_ext, mask_bias)
    h_att = _oproj(o, Wo, h2, ln1_g.reshape(1, D), ln1_b.reshape(1, D))
    dest2d, be2d, tot2d = _gate(h_att, Wg, bg.reshape(1, E))
    dest = dest2d.reshape(M)
    be = be2d.reshape(NBLK)
    tot = tot2d.reshape(1)
    x_sorted = _dispatch(h_att, dest)
    y_sorted = _ffn(be, tot, x_sorted, W1, b1, W2, b2)
    y = _combine(y_sorted, dest)
    out = _final(h_att, y, lnm_g.reshape(1, D), lnm_b.reshape(1, D),
                 ln2_g.reshape(1, D), ln2_b.reshape(1, D))
    return out.reshape(1, M, D)
